# Initial kernel scaffold; baseline (speedup 1.0000x reference)
import jax
import jax.numpy as jnp
from jax.experimental import pallas as pl


def kernel(node_features, edge_index, edge_features, batch_indices, W_enc, b_enc, W1, b1, W2, b2, gw_ih, gw_hh, gb_ih, gb_hh, lw_ih, lw_hh, lb_ih, lb_hh, W3, b3, W4, b4):
    raise NotImplementedError("write your pallas kernel here")


# speedup vs baseline: 2.4539x; 2.4539x over previous
import functools

import jax
import jax.numpy as jnp
from jax import lax
from jax.experimental import pallas as pl
from jax.experimental.pallas import tpu as pltpu
from jax.experimental.pallas import tpu_sc as plsc

H = 32
NG = 256
N_NODES = 10000
E_EDGES = 160000
NC, NS = 2, 16
NW = NC * NS
E_PAD = 163840
PER_W = E_PAD // NW
CHUNK = 128
NCHUNK = PER_W // CHUNK
ACC_R = 10112
ROWS_SUB = ACC_R // NS
DUMMY_ROW = 10104


GB = 4
G_EDGES = GB * CHUNK
G_ITERS = PER_W // G_EDGES


@functools.lru_cache(maxsize=None)
def _sc_gather_kernel():
    mesh = plsc.VectorSubcoreMesh(core_axis_name="c", subcore_axis_name="s",
                                  num_cores=NC, num_subcores=NS)

    @functools.partial(
        pl.kernel,
        out_type=jax.ShapeDtypeStruct((E_PAD, 128), jnp.float32),
        mesh=mesh,
        scratch_types=[
            pltpu.VMEM((GB, CHUNK), jnp.int32),
            pltpu.VMEM((G_EDGES, 128), jnp.float32),
            pltpu.SemaphoreType.DMA,
        ],
    )
    def body_fn(h_hbm, idx2_hbm, out_hbm, idx_v, rows_v, sem):
        wid = lax.axis_index("s") * NC + lax.axis_index("c")
        base_row = wid * (PER_W // CHUNK)

        def body(i, carry):
            r = pl.multiple_of(base_row + i * GB, GB)
            pltpu.sync_copy(idx2_hbm.at[pl.ds(r, GB)], idx_v)
            for j in range(GB):
                pltpu.async_copy(
                    h_hbm.at[idx_v.at[j]],
                    rows_v.at[pl.ds(j * CHUNK, CHUNK)], sem).wait()
            off = pl.multiple_of((base_row + i * GB) * CHUNK, G_EDGES)
            pltpu.sync_copy(rows_v, out_hbm.at[pl.ds(off, G_EDGES)])
            return carry

        lax.fori_loop(0, G_ITERS, body, 0)

    return body_fn


def _sc_gather(h, idx2):
    return _sc_gather_kernel()(h, idx2)


SB = 8
S_EDGES = SB * CHUNK


@functools.lru_cache(maxsize=None)
def _sc_scatter_kernel():
    mesh = plsc.VectorSubcoreMesh(core_axis_name="c", subcore_axis_name="s",
                                  num_cores=NC, num_subcores=NS)

    @functools.partial(
        pl.kernel,
        out_type=jax.ShapeDtypeStruct((NC, ACC_R, 128), jnp.float32),
        mesh=mesh,
        scratch_types=[
            pltpu.VMEM((SB, CHUNK), jnp.int32),
            pltpu.VMEM((CHUNK, 128), jnp.float32),
            pltpu.VMEM_SHARED((ACC_R, 128), jnp.float32),
            pltpu.SemaphoreType.DMA,
        ],
    )
    def body_fn(msg_hbm, idx2_hbm, zero_hbm, out_hbm, idx_v, rows_v, acc_sh, sem):
        cid = lax.axis_index("c")
        sid = lax.axis_index("s")
        wid = sid * NC + cid
        r0 = sid * ROWS_SUB
        pltpu.sync_copy(zero_hbm.at[pl.ds(r0, ROWS_SUB)],
                        acc_sh.at[pl.ds(r0, ROWS_SUB)])
        plsc.subcore_barrier()
        base_row = wid * (PER_W // CHUNK)

        def body(i, carry):
            r = pl.multiple_of(base_row + i * SB, SB)
            pltpu.sync_copy(idx2_hbm.at[pl.ds(r, SB)], idx_v)
            for j in range(SB):
                off = pl.multiple_of((base_row + i * SB + j) * CHUNK, CHUNK)
                pltpu.sync_copy(msg_hbm.at[pl.ds(off, CHUNK)], rows_v)
                pltpu.sync_copy(rows_v, acc_sh.at[idx_v.at[j]], add=True)
            return carry

        lax.fori_loop(0, NCHUNK // SB, body, 0)
        plsc.subcore_barrier()
        pltpu.sync_copy(acc_sh.at[pl.ds(r0, ROWS_SUB)],
                        out_hbm.at[cid, pl.ds(r0, ROWS_SUB)])

    return body_fn


def _sc_scatter(msg, idx2, zeros_acc):
    return _sc_scatter_kernel()(msg, idx2, zeros_acc)


def _encode_body(nf_ref, w_ref, b_ref, out_ref):
    res = (jnp.dot(nf_ref[...], w_ref[...], preferred_element_type=jnp.float32)
           + b_ref[...])
    out_ref[...] = jnp.pad(res, ((0, 0), (0, 128 - H)))


def _edgenet_body(ef_ref, w1_ref, b1_ref, out_ref):
    t = jnp.dot(ef_ref[...], w1_ref[...], preferred_element_type=jnp.float32)
    out_ref[...] = jnp.maximum(t + b1_ref[...], 0.0)


def _msg_body(t_ref, hs_ref, w2_ref, b2_ref, s_ref, out_ref):
    a = jnp.dot(t_ref[...], w2_ref[...], preferred_element_type=jnp.float32)
    a = a + b2_ref[...]
    ht = jnp.tile(hs_ref[:, :H], (1, H))
    p = a * ht
    m = jnp.dot(p, s_ref[...], preferred_element_type=jnp.float32)
    out_ref[...] = jnp.pad(m, ((0, 0), (0, 128 - H)))


def _gru_body(p_ref, h_ref, gih_ref, ghh_ref, bih_ref, bhh_ref, out_ref):
    m = p_ref[0, :N_NODES, :H] + p_ref[1, :N_NODES, :H]
    h = h_ref[:, :H]
    gi = jnp.dot(m, gih_ref[...], preferred_element_type=jnp.float32) + bih_ref[...]
    gh = jnp.dot(h, ghh_ref[...], preferred_element_type=jnp.float32) + bhh_ref[...]
    r = jax.nn.sigmoid(gi[:, :H] + gh[:, :H])
    z = jax.nn.sigmoid(gi[:, H:2 * H] + gh[:, H:2 * H])
    n = jnp.tanh(gi[:, 2 * H:] + r * gh[:, 2 * H:])
    out_ref[...] = jnp.pad((1.0 - z) * n + z * h, ((0, 0), (0, 128 - H)))


def _s2s_body(h_ref, b_ref, lih_ref, lhh_ref, lbih_ref, lbhh_ref,
              w3_ref, b3_ref, w4_ref, b4_ref, out_ref):
    h = h_ref[:, :H]
    bidx = b_ref[...]
    gids = lax.broadcasted_iota(jnp.int32, (N_NODES, NG), 1)
    onehot = (gids == bidx).astype(jnp.float32)
    gids_t = lax.broadcasted_iota(jnp.int32, (NG, N_NODES), 0)
    onehot_t = (gids_t == bidx.reshape(1, N_NODES)).astype(jnp.float32)

    s2s_h = jnp.zeros((NG, H), jnp.float32)
    s2s_c = jnp.zeros((NG, H), jnp.float32)
    r_out = jnp.zeros((NG, H), jnp.float32)
    for _ in range(4):
        q = jnp.dot(onehot, s2s_h, preferred_element_type=jnp.float32)
        e = jnp.sum(h * q, axis=1, keepdims=True)
        masked = jnp.where(onehot > 0.0, e, -1e30)
        e_max = jnp.max(masked, axis=0, keepdims=True)
        e_max_n = jnp.sum(onehot * e_max, axis=1, keepdims=True)
        ex = jnp.exp(e - e_max_n)
        denom = jnp.sum(onehot * ex, axis=0, keepdims=True)
        denom_n = jnp.sum(onehot * denom, axis=1, keepdims=True)
        a = ex / denom_n
        r_out = jnp.dot(onehot_t, a * h, preferred_element_type=jnp.float32)
        lstm_in = jnp.concatenate([s2s_h, r_out], axis=1)
        gates = (
            jnp.dot(lstm_in, lih_ref[...], preferred_element_type=jnp.float32)
            + lbih_ref[...]
            + jnp.dot(s2s_h, lhh_ref[...], preferred_element_type=jnp.float32)
            + lbhh_ref[...]
        )
        ii = jax.nn.sigmoid(gates[:, :H])
        ff = jax.nn.sigmoid(gates[:, H:2 * H])
        gg = jnp.tanh(gates[:, 2 * H:3 * H])
        oo = jax.nn.sigmoid(gates[:, 3 * H:])
        s2s_c = ff * s2s_c + ii * gg
        s2s_h = oo * jnp.tanh(s2s_c)

    ge = jnp.concatenate([s2s_h, r_out], axis=1)
    hid = jnp.maximum(
        jnp.dot(ge, w3_ref[...], preferred_element_type=jnp.float32) + b3_ref[...],
        0.0)
    out_ref[...] = (
        jnp.dot(hid, w4_ref[...], preferred_element_type=jnp.float32) + b4_ref[...]
    )


def _full(shape, dtype=jnp.float32):
    return pl.BlockSpec(shape, lambda *_: tuple(0 for _ in shape))


EB = 512


def kernel(node_features, edge_index, edge_features, batch_indices,
           W_enc, b_enc, W1, b1, W2, b2, gw_ih, gw_hh, gb_ih, gb_hh,
           lw_ih, lw_hh, lb_ih, lb_hh, W3, b3, W4, b4):
    f32 = jnp.float32
    npad = E_PAD - E_EDGES
    src_p = jnp.concatenate(
        [edge_index[0], jnp.zeros((npad,), jnp.int32)]).reshape(-1, CHUNK)
    dst_p = jnp.concatenate(
        [edge_index[1], jnp.full((npad,), DUMMY_ROW, jnp.int32)]).reshape(-1, CHUNK)
    ef_p = jnp.pad(edge_features, ((0, npad), (0, 0)))
    zeros_acc = jnp.zeros((ACC_R, 128), f32)
    s_mat = (jnp.arange(H * H)[:, None] // H == jnp.arange(H)[None, :]).astype(f32)

    b_enc2 = b_enc.reshape(1, H)
    b1_2 = b1.reshape(1, 64)
    b2_2 = b2.reshape(1, H * H)
    gih_t, ghh_t = gw_ih.T, gw_hh.T
    bih_2, bhh_2 = gb_ih.reshape(1, 3 * H), gb_hh.reshape(1, 3 * H)
    lih_t, lhh_t = lw_ih.T, lw_hh.T
    lbih_2, lbhh_2 = lb_ih.reshape(1, 4 * H), lb_hh.reshape(1, 4 * H)
    b3_2 = b3.reshape(1, H)
    w4_p = jnp.pad(W4, ((0, 0), (0, 128 - 3)))
    b4_p = jnp.pad(b4, (0, 128 - 3)).reshape(1, 128)
    bidx_2 = batch_indices.reshape(N_NODES, 1)

    h = pl.pallas_call(
        _encode_body,
        out_shape=jax.ShapeDtypeStruct((N_NODES, 128), f32),
        in_specs=[_full((N_NODES, 128)), _full((128, H)), _full((1, H))],
        out_specs=_full((N_NODES, 128)),
    )(node_features, W_enc, b_enc2)

    t = pl.pallas_call(
        _edgenet_body,
        grid=(E_PAD // 8192,),
        out_shape=jax.ShapeDtypeStruct((E_PAD, 64), f32),
        in_specs=[
            pl.BlockSpec((8192, 16), lambda i: (i, 0)),
            pl.BlockSpec((16, 64), lambda i: (0, 0)),
            pl.BlockSpec((1, 64), lambda i: (0, 0)),
        ],
        out_specs=pl.BlockSpec((8192, 64), lambda i: (i, 0)),
    )(ef_p, W1, b1_2)

    msg_call = pl.pallas_call(
        _msg_body,
        grid=(E_PAD // EB,),
        out_shape=jax.ShapeDtypeStruct((E_PAD, 128), f32),
        in_specs=[
            pl.BlockSpec((EB, 64), lambda i: (i, 0)),
            pl.BlockSpec((EB, 128), lambda i: (i, 0)),
            pl.BlockSpec((64, H * H), lambda i: (0, 0)),
            pl.BlockSpec((1, H * H), lambda i: (0, 0)),
            pl.BlockSpec((H * H, H), lambda i: (0, 0)),
        ],
        out_specs=pl.BlockSpec((EB, 128), lambda i: (i, 0)),
    )

    gru_call = pl.pallas_call(
        _gru_body,
        out_shape=jax.ShapeDtypeStruct((N_NODES, 128), f32),
        in_specs=[
            _full((NC, ACC_R, 128)), _full((N_NODES, 128)),
            _full((H, 3 * H)), _full((H, 3 * H)),
            _full((1, 3 * H)), _full((1, 3 * H)),
        ],
        out_specs=_full((N_NODES, 128)),
    )

    for _ in range(3):
        h_src = _sc_gather(h, src_p)
        msg = msg_call(t, h_src, W2, b2_2, s_mat)
        parts = _sc_scatter(msg, dst_p, zeros_acc)
        h = gru_call(parts, h, gih_t, ghh_t, bih_2, bhh_2)

    out = pl.pallas_call(
        _s2s_body,
        out_shape=jax.ShapeDtypeStruct((NG, 128), f32),
        in_specs=[
            _full((N_NODES, 128)), _full((N_NODES, 1)),
            _full((2 * H, 4 * H)), _full((H, 4 * H)),
            _full((1, 4 * H)), _full((1, 4 * H)),
            _full((2 * H, H)), _full((1, H)),
            _full((H, 128)), _full((1, 128)),
        ],
        out_specs=_full((NG, 128)),
    )(h, bidx_2, lih_t, lhh_t, lbih_2, lbhh_2, W3, b3_2, w4_p, b4_p)

    return out[:, :3]


# speedup vs baseline: 3.4442x; 1.4036x over previous
import functools

import jax
import jax.numpy as jnp
from jax import lax
from jax.experimental import pallas as pl
from jax.experimental.pallas import tpu as pltpu
from jax.experimental.pallas import tpu_sc as plsc

H = 32
NG = 256
N_NODES = 10000
E_EDGES = 160000
NC, NS = 2, 16
NW = NC * NS
E_PAD = 163840
PER_W = E_PAD // NW
CHUNK = 128
NCHUNK = PER_W // CHUNK
ACC_R = 10112
ROWS_SUB = ACC_R // NS
DUMMY_ROW = 10104


N_ROWS_SUB = 632
GROUP = 4
NGROup = (PER_W // CHUNK) // GROUP


@functools.lru_cache(maxsize=None)
def _sc_gather_kernel():
    mesh = plsc.VectorSubcoreMesh(core_axis_name="c", subcore_axis_name="s",
                                  num_cores=NC, num_subcores=NS)

    @functools.partial(
        pl.kernel,
        out_type=jax.ShapeDtypeStruct((E_PAD, 128), jnp.float32),
        mesh=mesh,
        scratch_types=[
            pltpu.VMEM((PER_W // CHUNK, CHUNK), jnp.int32),
            pltpu.VMEM((CHUNK, 128), jnp.float32),
            pltpu.VMEM_SHARED((ACC_R, 128), jnp.float32),
            pltpu.SemaphoreType.DMA,
        ],
    )
    def body_fn(h_hbm, idx2_hbm, out_hbm, idx_v, rows_v, tbl_sh, sem):
        cid = lax.axis_index("c")
        sid = lax.axis_index("s")
        wid = sid * NC + cid
        base_row = wid * (PER_W // CHUNK)
        pltpu.sync_copy(idx2_hbm.at[pl.ds(pl.multiple_of(base_row, 8),
                                          PER_W // CHUNK)], idx_v)
        r0 = sid * N_ROWS_SUB
        pltpu.sync_copy(h_hbm.at[pl.ds(r0, N_ROWS_SUB)],
                        tbl_sh.at[pl.ds(r0, N_ROWS_SUB)])
        plsc.subcore_barrier()

        def body(g, carry):
            pltpu.async_copy(tbl_sh.at[idx_v.at[g]],
                             rows_v, sem).wait()
            off = pl.multiple_of((base_row + g) * CHUNK, CHUNK)
            pltpu.sync_copy(rows_v, out_hbm.at[pl.ds(off, CHUNK)])
            return carry

        lax.fori_loop(0, PER_W // CHUNK, body, 0)

    return body_fn


def _sc_gather(h, idx2):
    return _sc_gather_kernel()(h, idx2)


SB = 8
S_EDGES = SB * CHUNK


@functools.lru_cache(maxsize=None)
def _sc_scatter_kernel():
    mesh = plsc.VectorSubcoreMesh(core_axis_name="c", subcore_axis_name="s",
                                  num_cores=NC, num_subcores=NS)

    @functools.partial(
        pl.kernel,
        out_type=jax.ShapeDtypeStruct((NC, ACC_R, 128), jnp.float32),
        mesh=mesh,
        scratch_types=[
            pltpu.VMEM((SB, CHUNK), jnp.int32),
            pltpu.VMEM((CHUNK, 128), jnp.float32),
            pltpu.VMEM_SHARED((ACC_R, 128), jnp.float32),
            pltpu.SemaphoreType.DMA,
        ],
    )
    def body_fn(msg_hbm, idx2_hbm, zero_hbm, out_hbm, idx_v, rows_v, acc_sh, sem):
        cid = lax.axis_index("c")
        sid = lax.axis_index("s")
        wid = sid * NC + cid
        r0 = sid * ROWS_SUB
        pltpu.sync_copy(zero_hbm.at[pl.ds(r0, ROWS_SUB)],
                        acc_sh.at[pl.ds(r0, ROWS_SUB)])
        plsc.subcore_barrier()
        base_row = wid * (PER_W // CHUNK)

        def body(i, carry):
            r = pl.multiple_of(base_row + i * SB, SB)
            pltpu.sync_copy(idx2_hbm.at[pl.ds(r, SB)], idx_v)
            for j in range(SB):
                off = pl.multiple_of((base_row + i * SB + j) * CHUNK, CHUNK)
                pltpu.sync_copy(msg_hbm.at[pl.ds(off, CHUNK)], rows_v)
                pltpu.sync_copy(rows_v, acc_sh.at[idx_v.at[j]], add=True)
            return carry

        lax.fori_loop(0, NCHUNK // SB, body, 0)
        plsc.subcore_barrier()
        pltpu.sync_copy(acc_sh.at[pl.ds(r0, ROWS_SUB)],
                        out_hbm.at[cid, pl.ds(r0, ROWS_SUB)])

    return body_fn


def _sc_scatter(msg, idx2, zeros_acc):
    return _sc_scatter_kernel()(msg, idx2, zeros_acc)


def _encode_body(nf_ref, w_ref, b_ref, out_ref):
    res = (jnp.dot(nf_ref[...], w_ref[...], preferred_element_type=jnp.float32)
           + b_ref[...])
    out_ref[...] = jnp.pad(res, ((0, ACC_R - N_NODES), (0, 128 - H)))


def _edgenet_body(ef_ref, w1_ref, b1_ref, out_ref):
    t = jnp.dot(ef_ref[...], w1_ref[...], preferred_element_type=jnp.float32)
    out_ref[...] = jnp.maximum(t + b1_ref[...], 0.0)


def _msg_body(t_ref, hs_ref, w2_ref, b2_ref, s_ref, out_ref):
    a = jnp.dot(t_ref[...], w2_ref[...], preferred_element_type=jnp.float32)
    a = a + b2_ref[...]
    ht = jnp.tile(hs_ref[:, :H], (1, H))
    p = a * ht
    m = jnp.dot(p, s_ref[...], preferred_element_type=jnp.float32)
    out_ref[...] = jnp.pad(m, ((0, 0), (0, 128 - H)))


def _gru_body(p_ref, h_ref, gih_ref, ghh_ref, bih_ref, bhh_ref, out_ref):
    m = p_ref[0, :N_NODES, :H] + p_ref[1, :N_NODES, :H]
    h = h_ref[:N_NODES, :H]
    gi = jnp.dot(m, gih_ref[...], preferred_element_type=jnp.float32) + bih_ref[...]
    gh = jnp.dot(h, ghh_ref[...], preferred_element_type=jnp.float32) + bhh_ref[...]
    r = jax.nn.sigmoid(gi[:, :H] + gh[:, :H])
    z = jax.nn.sigmoid(gi[:, H:2 * H] + gh[:, H:2 * H])
    n = jnp.tanh(gi[:, 2 * H:] + r * gh[:, 2 * H:])
    out_ref[...] = jnp.pad((1.0 - z) * n + z * h,
                           ((0, ACC_R - N_NODES), (0, 128 - H)))


def _s2s_body(h_ref, b_ref, lih_ref, lhh_ref, lbih_ref, lbhh_ref,
              w3_ref, b3_ref, w4_ref, b4_ref, out_ref):
    h = h_ref[:N_NODES, :H]
    bidx = b_ref[...]
    gids = lax.broadcasted_iota(jnp.int32, (N_NODES, NG), 1)
    onehot = (gids == bidx).astype(jnp.float32)
    gids_t = lax.broadcasted_iota(jnp.int32, (NG, N_NODES), 0)
    onehot_t = (gids_t == bidx.reshape(1, N_NODES)).astype(jnp.float32)

    s2s_h = jnp.zeros((NG, H), jnp.float32)
    s2s_c = jnp.zeros((NG, H), jnp.float32)
    r_out = jnp.zeros((NG, H), jnp.float32)
    for _ in range(4):
        q = jnp.dot(onehot, s2s_h, preferred_element_type=jnp.float32)
        e = jnp.sum(h * q, axis=1, keepdims=True)
        masked = jnp.where(onehot > 0.0, e, -1e30)
        e_max = jnp.max(masked, axis=0, keepdims=True)
        e_max_n = jnp.sum(onehot * e_max, axis=1, keepdims=True)
        ex = jnp.exp(e - e_max_n)
        denom = jnp.sum(onehot * ex, axis=0, keepdims=True)
        denom_n = jnp.sum(onehot * denom, axis=1, keepdims=True)
        a = ex / denom_n
        r_out = jnp.dot(onehot_t, a * h, preferred_element_type=jnp.float32)
        lstm_in = jnp.concatenate([s2s_h, r_out], axis=1)
        gates = (
            jnp.dot(lstm_in, lih_ref[...], preferred_element_type=jnp.float32)
            + lbih_ref[...]
            + jnp.dot(s2s_h, lhh_ref[...], preferred_element_type=jnp.float32)
            + lbhh_ref[...]
        )
        ii = jax.nn.sigmoid(gates[:, :H])
        ff = jax.nn.sigmoid(gates[:, H:2 * H])
        gg = jnp.tanh(gates[:, 2 * H:3 * H])
        oo = jax.nn.sigmoid(gates[:, 3 * H:])
        s2s_c = ff * s2s_c + ii * gg
        s2s_h = oo * jnp.tanh(s2s_c)

    ge = jnp.concatenate([s2s_h, r_out], axis=1)
    hid = jnp.maximum(
        jnp.dot(ge, w3_ref[...], preferred_element_type=jnp.float32) + b3_ref[...],
        0.0)
    out_ref[...] = (
        jnp.dot(hid, w4_ref[...], preferred_element_type=jnp.float32) + b4_ref[...]
    )


def _full(shape, dtype=jnp.float32):
    return pl.BlockSpec(shape, lambda *_: tuple(0 for _ in shape))


EB = 512


def kernel(node_features, edge_index, edge_features, batch_indices,
           W_enc, b_enc, W1, b1, W2, b2, gw_ih, gw_hh, gb_ih, gb_hh,
           lw_ih, lw_hh, lb_ih, lb_hh, W3, b3, W4, b4):
    f32 = jnp.float32
    npad = E_PAD - E_EDGES
    src_p = jnp.concatenate(
        [edge_index[0], jnp.zeros((npad,), jnp.int32)]).reshape(-1, CHUNK)
    dst_p = jnp.concatenate(
        [edge_index[1], jnp.full((npad,), DUMMY_ROW, jnp.int32)]).reshape(-1, CHUNK)
    ef_p = jnp.pad(edge_features, ((0, npad), (0, 0)))
    zeros_acc = jnp.zeros((ACC_R, 128), f32)
    s_mat = (jnp.arange(H * H)[:, None] // H == jnp.arange(H)[None, :]).astype(f32)

    b_enc2 = b_enc.reshape(1, H)
    b1_2 = b1.reshape(1, 64)
    b2_2 = b2.reshape(1, H * H)
    gih_t, ghh_t = gw_ih.T, gw_hh.T
    bih_2, bhh_2 = gb_ih.reshape(1, 3 * H), gb_hh.reshape(1, 3 * H)
    lih_t, lhh_t = lw_ih.T, lw_hh.T
    lbih_2, lbhh_2 = lb_ih.reshape(1, 4 * H), lb_hh.reshape(1, 4 * H)
    b3_2 = b3.reshape(1, H)
    w4_p = jnp.pad(W4, ((0, 0), (0, 128 - 3)))
    b4_p = jnp.pad(b4, (0, 128 - 3)).reshape(1, 128)
    bidx_2 = batch_indices.reshape(N_NODES, 1)

    h = pl.pallas_call(
        _encode_body,
        out_shape=jax.ShapeDtypeStruct((ACC_R, 128), f32),
        in_specs=[_full((N_NODES, 128)), _full((128, H)), _full((1, H))],
        out_specs=_full((ACC_R, 128)),
    )(node_features, W_enc, b_enc2)

    t = pl.pallas_call(
        _edgenet_body,
        grid=(E_PAD // 8192,),
        out_shape=jax.ShapeDtypeStruct((E_PAD, 64), f32),
        in_specs=[
            pl.BlockSpec((8192, 16), lambda i: (i, 0)),
            pl.BlockSpec((16, 64), lambda i: (0, 0)),
            pl.BlockSpec((1, 64), lambda i: (0, 0)),
        ],
        out_specs=pl.BlockSpec((8192, 64), lambda i: (i, 0)),
    )(ef_p, W1, b1_2)

    msg_call = pl.pallas_call(
        _msg_body,
        grid=(E_PAD // EB,),
        out_shape=jax.ShapeDtypeStruct((E_PAD, 128), f32),
        in_specs=[
            pl.BlockSpec((EB, 64), lambda i: (i, 0)),
            pl.BlockSpec((EB, 128), lambda i: (i, 0)),
            pl.BlockSpec((64, H * H), lambda i: (0, 0)),
            pl.BlockSpec((1, H * H), lambda i: (0, 0)),
            pl.BlockSpec((H * H, H), lambda i: (0, 0)),
        ],
        out_specs=pl.BlockSpec((EB, 128), lambda i: (i, 0)),
    )

    gru_call = pl.pallas_call(
        _gru_body,
        out_shape=jax.ShapeDtypeStruct((ACC_R, 128), f32),
        in_specs=[
            _full((NC, ACC_R, 128)), _full((ACC_R, 128)),
            _full((H, 3 * H)), _full((H, 3 * H)),
            _full((1, 3 * H)), _full((1, 3 * H)),
        ],
        out_specs=_full((ACC_R, 128)),
    )

    for _ in range(3):
        h_src = _sc_gather(h, src_p)
        msg = msg_call(t, h_src, W2, b2_2, s_mat)
        parts = _sc_scatter(msg, dst_p, zeros_acc)
        h = gru_call(parts, h, gih_t, ghh_t, bih_2, bhh_2)

    out = pl.pallas_call(
        _s2s_body,
        out_shape=jax.ShapeDtypeStruct((NG, 128), f32),
        in_specs=[
            _full((ACC_R, 128)), _full((N_NODES, 1)),
            _full((2 * H, 4 * H)), _full((H, 4 * H)),
            _full((1, 4 * H)), _full((1, 4 * H)),
            _full((2 * H, H)), _full((1, H)),
            _full((H, 128)), _full((1, 128)),
        ],
        out_specs=_full((NG, 128)),
    )(h, bidx_2, lih_t, lhh_t, lbih_2, lbhh_2, W3, b3_2, w4_p, b4_p)

    return out[:, :3]
